# baseline (speedup 1.0000x reference)
import functools

import jax
import jax.numpy as jnp
from jax import lax
from jax.experimental import pallas as pl
from jax.experimental.pallas import tpu as pltpu
from jax.experimental.pallas import tpu_sc as plsc

_LANES = 16
_WINDOW = 512


def kernel(input_ids, table):
    B, S, H = input_ids.shape
    D = table.shape[-1]
    n_per_head = table.shape[0] // H
    N = B * S * H
    ids_flat = input_ids.reshape(1, N)

    mesh = plsc.VectorSubcoreMesh(
        core_axis_name="core", subcore_axis_name="subcore"
    )

    @functools.partial(
        pl.kernel,
        out_type=jax.ShapeDtypeStruct((N, D), table.dtype),
        mesh=mesh,
        scratch_types=[pltpu.VMEM((_WINDOW,), jnp.int32)],
        compiler_params=pltpu.CompilerParams(use_tc_tiling_on_sc=False),
    )
    def gather_kernel(ids_hbm, table_hbm, out_hbm, sidx):
        def body(i_vmem, o_vmem):
            offs = (
                lax.rem(
                    lax.iota(jnp.int32, _LANES),
                    jnp.full((_LANES,), H, jnp.int32),
                )
                * n_per_head
            )
            src = i_vmem.at[0]

            @pl.loop(0, _WINDOW, step=_LANES)
            def _(j):
                sidx[pl.ds(j, _LANES)] = src[pl.ds(j, _LANES)] + offs

            pltpu.sync_copy(table_hbm.at[sidx], o_vmem)

        pltpu.emit_pipeline(
            body,
            grid=(N // _WINDOW,),
            in_specs=[pl.BlockSpec((1, _WINDOW), index_map=lambda i: (0, i))],
            out_specs=[pl.BlockSpec((_WINDOW, D), index_map=lambda i: (i, 0))],
            core_axis_name=("core", "subcore"),
            dimension_semantics=(pltpu.PARALLEL,),
        )(ids_hbm, out_hbm)

    out = gather_kernel(ids_flat, table)
    return out.reshape(B, S, H, D)
